# baseline (speedup 1.0000x reference)
import functools

import jax
import jax.numpy as jnp
from jax import lax
from jax.experimental import pallas as pl
from jax.experimental.pallas import tpu as pltpu
from jax.experimental.pallas import tpu_sc as plsc

N = 10000
E = 320000
H = 128
NCLASS = 40
NP = 10112
ROWS_PER_TILE = NP // 16
NPD = 10240
DEG_ROWS_PER_TILE = NPD // 16
EP = 331776
PAD = EP - E
KE = 128
EPT = EP // 16
NCHUNK = EPT // KE
KD = 128
HALF_E = EP // 2
DPT = HALF_E // 16
DCHUNK = DPT // KD

_mesh = plsc.VectorSubcoreMesh(core_axis_name="c", subcore_axis_name="s")


def _deg_body(colp, out, cbuf, ones, zv, degS, sem):
    c = lax.axis_index("c")
    s = lax.axis_index("s")
    for j in range(8):
        ones[pl.ds(16 * j, 16)] = jnp.ones((16,), jnp.float32)

    def zb(i, cy):
        zv[pl.ds(i * 16, 16)] = jnp.zeros((16,), jnp.float32)
        return cy

    lax.fori_loop(0, DEG_ROWS_PER_TILE // 16, zb, 0)
    pltpu.sync_copy(zv, degS.at[pl.ds(s * DEG_ROWS_PER_TILE, DEG_ROWS_PER_TILE)])
    plsc.subcore_barrier()

    def chunk(g, cy):
        base = c * HALF_E + s * DPT + g * KD
        pltpu.sync_copy(colp.at[pl.ds(base, KD)], cbuf.at[0])
        pltpu.sync_copy(ones, degS.at[cbuf.at[0]], add=True)
        return cy

    lax.fori_loop(0, DCHUNK, chunk, 0)
    plsc.subcore_barrier()

    @pl.when(c == 0)
    def _():
        pltpu.sync_copy(degS.at[pl.ds(s * DEG_ROWS_PER_TILE, DEG_ROWS_PER_TILE)],
                        out.at[0].at[pl.ds(s * DEG_ROWS_PER_TILE, DEG_ROWS_PER_TILE)])

    @pl.when(c == 1)
    def _():
        pltpu.sync_copy(degS.at[pl.ds(s * DEG_ROWS_PER_TILE, DEG_ROWS_PER_TILE)],
                        out.at[1].at[pl.ds(s * DEG_ROWS_PER_TILE, DEG_ROWS_PER_TILE)])


_deg_call = pl.kernel(
    _deg_body,
    out_type=jax.ShapeDtypeStruct((2, NPD), jnp.float32),
    mesh=_mesh,
    scratch_types=[
        pltpu.VMEM((1, KD), jnp.int32),
        pltpu.VMEM((KD,), jnp.float32),
        pltpu.VMEM((DEG_ROWS_PER_TILE,), jnp.float32),
        pltpu.VMEM_SHARED((NPD,), jnp.float32),
        pltpu.SemaphoreType.DMA,
    ],
)


def _edge_body(rowp, colp, XWm, xwn, zrows, out,
               rbuf, cbuf, sbuf, dbuf, accS,
               sem_s0, sem_s1, sem_d, sem_i0, sem_i1, sem_i2,
               sem_c0, sem_c1):
    c = lax.axis_index("c")
    s = lax.axis_index("s")
    sem_s = (sem_s0, sem_s1)
    sem_i = (sem_i0, sem_i1, sem_i2)
    sem_c = (sem_c0, sem_c1)

    pltpu.sync_copy(zrows.at[pl.ds(s * ROWS_PER_TILE, ROWS_PER_TILE)],
                    accS.at[pl.ds(s * ROWS_PER_TILE, ROWS_PER_TILE)])
    plsc.subcore_barrier()

    def idxload(g, q):
        base = s * EPT + g * KE
        pltpu.async_copy(rowp.at[pl.ds(base, KE)], rbuf.at[q], sem_i[q])
        pltpu.async_copy(colp.at[pl.ds(base, KE)], cbuf.at[q], sem_i[q])

    def idxwait(g, q):
        base = s * EPT + g * KE
        pltpu.make_async_copy(rowp.at[pl.ds(base, KE)], rbuf.at[q],
                              sem_i[q]).wait()
        pltpu.make_async_copy(colp.at[pl.ds(base, KE)], cbuf.at[q],
                              sem_i[q]).wait()

    def run(table, with_diff):
        def rowg(q, p):
            pltpu.async_copy(table.at[rbuf.at[q]], sbuf.at[p], sem_s[p])

        def colg(q):
            pltpu.async_copy(XWm.at[cbuf.at[q]], dbuf, sem_d)

        def scat_wait(p, q):
            pltpu.make_async_copy(sbuf.at[p], accS.at[cbuf.at[q]],
                                  sem_c[p]).wait()

        def consume(g, p, q):
            pltpu.make_async_copy(table.at[rbuf.at[q]], sbuf.at[p],
                                  sem_s[p]).wait()
            if with_diff:
                pltpu.make_async_copy(XWm.at[cbuf.at[q]], dbuf, sem_d).wait()

                @plsc.parallel_loop(0, KE, 1, unroll=4)
                def _(i):
                    for j in range(8):
                        sl = pl.ds(16 * j, 16)
                        sbuf[p, i, sl] = jnp.maximum(
                            sbuf[p, i, sl] - dbuf[i, sl], 0.0)

                @pl.when(g + 1 < NCHUNK)
                def _():
                    colg((g + 1) % 3)

            pltpu.async_copy(sbuf.at[p], accS.at[cbuf.at[q]], sem_c[p],
                             add=True)

        idxload(0, 0)
        idxload(1, 1)
        idxwait(0, 0)
        rowg(0, 0)
        if with_diff:
            colg(0)

        def six(ii, cy):
            g0 = 6 * ii
            for u in range(6):
                g = g0 + u
                p, q = u % 2, u % 3

                @pl.when(g >= 1)
                def _():
                    scat_wait(1 - p, (q + 2) % 3)

                @pl.when(g + 2 < NCHUNK)
                def _():
                    idxload(g + 2, (q + 2) % 3)

                @pl.when(g + 1 < NCHUNK)
                def _():
                    idxwait(g + 1, (q + 1) % 3)
                    rowg((q + 1) % 3, 1 - p)

                consume(g, p, q)
            return cy

        lax.fori_loop(0, NCHUNK // 6, six, 0)
        scat_wait((NCHUNK - 1) % 2, (NCHUNK - 1) % 3)

    @pl.when(c == 0)
    def _():
        run(XWm, True)

    @pl.when(c == 1)
    def _():
        run(xwn, False)

    plsc.subcore_barrier()

    @pl.when(c == 0)
    def _():
        pltpu.sync_copy(accS.at[pl.ds(s * ROWS_PER_TILE, ROWS_PER_TILE)],
                        out.at[0].at[pl.ds(s * ROWS_PER_TILE, ROWS_PER_TILE)])

    @pl.when(c == 1)
    def _():
        pltpu.sync_copy(accS.at[pl.ds(s * ROWS_PER_TILE, ROWS_PER_TILE)],
                        out.at[1].at[pl.ds(s * ROWS_PER_TILE, ROWS_PER_TILE)])


_edge_call = pl.kernel(
    _edge_body,
    out_type=jax.ShapeDtypeStruct((2, NP, H), jnp.float32),
    mesh=_mesh,
    scratch_types=[
        pltpu.VMEM((3, KE), jnp.int32),
        pltpu.VMEM((3, KE), jnp.int32),
        pltpu.VMEM((2, KE, H), jnp.float32),
        pltpu.VMEM((KE, H), jnp.float32),
        pltpu.VMEM_SHARED((NP, H), jnp.float32),
        pltpu.SemaphoreType.DMA,
        pltpu.SemaphoreType.DMA,
        pltpu.SemaphoreType.DMA,
        pltpu.SemaphoreType.DMA,
        pltpu.SemaphoreType.DMA,
        pltpu.SemaphoreType.DMA,
        pltpu.SemaphoreType.DMA,
        pltpu.SemaphoreType.DMA,
    ],
)


R = 1000
GRID = N // R
_f32 = jnp.float32


def _dotT(a, b):
    return lax.dot_general(a, b, (((1,), (1,)), ((), ())),
                           preferred_element_type=_f32)


def _dot(a, b):
    return lax.dot_general(a, b, (((1,), (0,)), ((), ())),
                           preferred_element_type=_f32)


def _dinv_of(degt):
    return lax.rsqrt(1.0 + degt[:, 0:1] + degt[:, 1:2])


def _emit_tables(X, dinv, Wm, Wc, Wr, br,
                 xw_ref, res_ref, XWm_ref, xwn_ref):
    XWm = _dot(X, Wm[...])
    xw = _dotT(X, Wc[...])
    xw_ref[...] = xw
    res_ref[...] = -(_dotT(xw, Wr[...]) + br[...])
    XWm_ref[...] = XWm
    xwn_ref[...] = xw * dinv


def _node_update(acc_ref, X, xw, res, dinv, bc, lam):
    T = acc_ref[0]
    S = acc_ref[1]
    conv = dinv * S + (dinv * dinv) * xw + bc[...]
    return jnp.maximum(conv + res, 0.0) + lam * (X * T)


def _pre_body(x_ref, degt_ref, We, be, Wm, Wc, Wr, br,
              X0_ref, xw_ref, res_ref, XWm_ref, xwn_ref):
    X0 = jnp.maximum(_dotT(x_ref[...], We[...]) + be[...], 0.0)
    X0_ref[...] = X0
    dinv = _dinv_of(degt_ref[...])
    _emit_tables(X0, dinv, Wm, Wc, Wr, br, xw_ref, res_ref, XWm_ref, xwn_ref)


def _mid_body(acc_ref, X_ref, xwin_ref, resin_ref, degt_ref,
              Wm, Wc, Wr, br, bc, lam_ref,
              X1_ref, xw_ref, res_ref, XWm_ref, xwn_ref):
    dinv = _dinv_of(degt_ref[...])
    X1 = _node_update(acc_ref, X_ref[...], xwin_ref[...], resin_ref[...],
                      dinv, bc, lam_ref[0, 0])
    X1_ref[...] = X1
    _emit_tables(X1, dinv, Wm, Wc, Wr, br, xw_ref, res_ref, XWm_ref, xwn_ref)


def _post_body(acc_ref, X_ref, xwin_ref, resin_ref, degt_ref,
               Wd, bd, bc, lam_ref, out_ref):
    dinv = _dinv_of(degt_ref[...])
    X2 = _node_update(acc_ref, X_ref[...], xwin_ref[...], resin_ref[...],
                      dinv, bc, lam_ref[0, 0])
    out_ref[...] = _dotT(X2, Wd[...]) + bd[...]


def _row_spec(w):
    return pl.BlockSpec((R, w), lambda i: (i, 0))


def _full_spec(shape):
    return pl.BlockSpec(shape, lambda i: tuple(0 for _ in shape))


_degt_spec = pl.BlockSpec((R, 2), lambda i: (i, 0))
_acc_spec = pl.BlockSpec((2, R, H), lambda i: (0, i, 0))
_lam_spec = pl.BlockSpec((1, 1), lambda i: (0, 0), memory_space=pltpu.SMEM)

_table_out_shapes = [
    jax.ShapeDtypeStruct((N, H), _f32),
    jax.ShapeDtypeStruct((N, H), _f32),
    jax.ShapeDtypeStruct((NP, H), _f32),
    jax.ShapeDtypeStruct((NP, H), _f32),
]
_table_out_specs = [_row_spec(H), _row_spec(H), _row_spec(H), _row_spec(H)]

_pre = pl.pallas_call(
    _pre_body,
    grid=(GRID,),
    in_specs=[_row_spec(H), _degt_spec, _full_spec((H, H)), _full_spec((1, H)),
              _full_spec((H, H)), _full_spec((H, H)), _full_spec((H, H)),
              _full_spec((1, H))],
    out_specs=[_row_spec(H)] + _table_out_specs,
    out_shape=[jax.ShapeDtypeStruct((N, H), _f32)] + _table_out_shapes,
)

_mid = pl.pallas_call(
    _mid_body,
    grid=(GRID,),
    in_specs=[_acc_spec, _row_spec(H), _row_spec(H), _row_spec(H), _degt_spec,
              _full_spec((H, H)), _full_spec((H, H)), _full_spec((H, H)),
              _full_spec((1, H)), _full_spec((1, H)), _lam_spec],
    out_specs=[_row_spec(H)] + _table_out_specs,
    out_shape=[jax.ShapeDtypeStruct((N, H), _f32)] + _table_out_shapes,
)

_post = pl.pallas_call(
    _post_body,
    grid=(GRID,),
    in_specs=[_acc_spec, _row_spec(H), _row_spec(H), _row_spec(H), _degt_spec,
              _full_spec((NCLASS, H)), _full_spec((1, NCLASS)),
              _full_spec((1, H)), _lam_spec],
    out_specs=[_row_spec(NCLASS)],
    out_shape=[jax.ShapeDtypeStruct((N, NCLASS), _f32)],
)


def kernel(x, edge_index, W_enc, b_enc, W_conv, b_conv, W_res, b_res,
           W_dec, b_dec, weight_mlp, lamda1):
    row = edge_index[0].astype(jnp.int32)
    col = edge_index[1].astype(jnp.int32)
    rowp = jnp.concatenate([row, jnp.arange(PAD, dtype=jnp.int32) % N])
    colp = jnp.concatenate([col, N + (jnp.arange(PAD, dtype=jnp.int32) % 16)])
    zrows = jnp.zeros((NP, H), _f32)

    be = b_enc.reshape(1, H)
    br = b_res.reshape(1, H)
    bc = b_conv.reshape(1, H)
    bd = b_dec.reshape(1, NCLASS)
    lam = lamda1.reshape(1, 1)

    degt = _deg_call(colp).T

    X0, xw1, res1, XWm1, xwn1 = _pre(x, degt, W_enc, be, weight_mlp,
                                     W_conv, W_res, br)
    acc1 = _edge_call(rowp, colp, XWm1, xwn1, zrows)
    X1, xw2, res2, XWm2, xwn2 = _mid(acc1, X0, xw1, res1, degt, weight_mlp,
                                     W_conv, W_res, br, bc, lam)
    acc2 = _edge_call(rowp, colp, XWm2, xwn2, zrows)
    (out,) = _post(acc2, X1, xw2, res2, degt, W_dec, bd, bc, lam)
    return out


# speedup vs baseline: 1.2313x; 1.2313x over previous
import functools

import jax
import jax.numpy as jnp
from jax import lax
from jax.experimental import pallas as pl
from jax.experimental.pallas import tpu as pltpu
from jax.experimental.pallas import tpu_sc as plsc

N = 10000
E = 320000
H = 128
NCLASS = 40
NP = 10112
ROWS_PER_TILE = NP // 16
NPD = 10240
DEG_ROWS_PER_TILE = NPD // 16
EP = 344064
PAD = EP - E
KE = 128
HALF_E = EP // 2
PPT = HALF_E // 16
NCHUNK = PPT // KE
KD = 128
DPT = HALF_E // 16
DCHUNK = DPT // KD

_mesh = plsc.VectorSubcoreMesh(core_axis_name="c", subcore_axis_name="s")


def _deg_body(colp, out, cbuf, ones, zv, degS, sem):
    c = lax.axis_index("c")
    s = lax.axis_index("s")
    for j in range(8):
        ones[pl.ds(16 * j, 16)] = jnp.ones((16,), jnp.float32)

    def zb(i, cy):
        zv[pl.ds(i * 16, 16)] = jnp.zeros((16,), jnp.float32)
        return cy

    lax.fori_loop(0, DEG_ROWS_PER_TILE // 16, zb, 0)
    pltpu.sync_copy(zv, degS.at[pl.ds(s * DEG_ROWS_PER_TILE, DEG_ROWS_PER_TILE)])
    plsc.subcore_barrier()

    def chunk(g, cy):
        base = c * HALF_E + s * DPT + g * KD
        pltpu.sync_copy(colp.at[pl.ds(base, KD)], cbuf.at[0])
        pltpu.sync_copy(ones, degS.at[cbuf.at[0]], add=True)
        return cy

    lax.fori_loop(0, DCHUNK, chunk, 0)
    plsc.subcore_barrier()

    @pl.when(c == 0)
    def _():
        pltpu.sync_copy(degS.at[pl.ds(s * DEG_ROWS_PER_TILE, DEG_ROWS_PER_TILE)],
                        out.at[0].at[pl.ds(s * DEG_ROWS_PER_TILE, DEG_ROWS_PER_TILE)])

    @pl.when(c == 1)
    def _():
        pltpu.sync_copy(degS.at[pl.ds(s * DEG_ROWS_PER_TILE, DEG_ROWS_PER_TILE)],
                        out.at[1].at[pl.ds(s * DEG_ROWS_PER_TILE, DEG_ROWS_PER_TILE)])


_deg_call = pl.kernel(
    _deg_body,
    out_type=jax.ShapeDtypeStruct((2, NPD), jnp.float32),
    mesh=_mesh,
    scratch_types=[
        pltpu.VMEM((1, KD), jnp.int32),
        pltpu.VMEM((KD,), jnp.float32),
        pltpu.VMEM((DEG_ROWS_PER_TILE,), jnp.float32),
        pltpu.VMEM_SHARED((NPD,), jnp.float32),
        pltpu.SemaphoreType.DMA,
    ],
)


def _edge_body(rowp, colp, XWm, xwn, zrows, Tout, Sout,
               rbuf, cbuf, sbuf, dbuf, accS,
               sem_s0, sem_s1, sem_d, sem_i0, sem_i1, sem_i2,
               sem_c0, sem_c1):
    c = lax.axis_index("c")
    s = lax.axis_index("s")
    sem_s = (sem_s0, sem_s1)
    sem_i = (sem_i0, sem_i1, sem_i2)
    sem_c = (sem_c0, sem_c1)

    def zero_acc():
        pltpu.sync_copy(zrows.at[pl.ds(s * ROWS_PER_TILE, ROWS_PER_TILE)],
                        accS.at[pl.ds(s * ROWS_PER_TILE, ROWS_PER_TILE)])

    def copy_out(dst):
        pltpu.sync_copy(accS.at[pl.ds(s * ROWS_PER_TILE, ROWS_PER_TILE)],
                        dst.at[pl.ds(s * ROWS_PER_TILE, ROWS_PER_TILE)])

    def run(table, with_diff):
        base0 = c * HALF_E + s * PPT

        def idxload(g, q):
            base = base0 + g * KE
            pltpu.async_copy(rowp.at[pl.ds(base, KE)], rbuf.at[q], sem_i[q])
            pltpu.async_copy(colp.at[pl.ds(base, KE)], cbuf.at[q], sem_i[q])

        def idxwait(g, q):
            base = base0 + g * KE
            pltpu.make_async_copy(rowp.at[pl.ds(base, KE)], rbuf.at[q],
                                  sem_i[q]).wait()
            pltpu.make_async_copy(colp.at[pl.ds(base, KE)], cbuf.at[q],
                                  sem_i[q]).wait()

        def rowg(q, p):
            pltpu.async_copy(table.at[rbuf.at[q]], sbuf.at[p], sem_s[p])

        def colg(q):
            pltpu.async_copy(XWm.at[cbuf.at[q]], dbuf, sem_d)

        def scat_wait(p, q):
            pltpu.make_async_copy(sbuf.at[p], accS.at[cbuf.at[q]],
                                  sem_c[p]).wait()

        def consume(g, p, q):
            pltpu.make_async_copy(table.at[rbuf.at[q]], sbuf.at[p],
                                  sem_s[p]).wait()
            if with_diff:
                pltpu.make_async_copy(XWm.at[cbuf.at[q]], dbuf, sem_d).wait()

                @plsc.parallel_loop(0, KE, 1, unroll=4)
                def _(i):
                    for j in range(8):
                        sl = pl.ds(16 * j, 16)
                        sbuf[p, i, sl] = jnp.maximum(
                            sbuf[p, i, sl] - dbuf[i, sl], 0.0)

                @pl.when(g + 1 < NCHUNK)
                def _():
                    colg((g + 1) % 3)

            pltpu.async_copy(sbuf.at[p], accS.at[cbuf.at[q]], sem_c[p],
                             add=True)

        idxload(0, 0)
        idxload(1, 1)
        idxwait(0, 0)
        rowg(0, 0)
        if with_diff:
            colg(0)

        def six(ii, cy):
            g0 = 6 * ii
            for u in range(6):
                g = g0 + u
                p, q = u % 2, u % 3

                @pl.when(g >= 1)
                def _():
                    scat_wait(1 - p, (q + 2) % 3)

                @pl.when(g + 2 < NCHUNK)
                def _():
                    idxload(g + 2, (q + 2) % 3)

                @pl.when(g + 1 < NCHUNK)
                def _():
                    idxwait(g + 1, (q + 1) % 3)
                    rowg((q + 1) % 3, 1 - p)

                consume(g, p, q)
            return cy

        lax.fori_loop(0, NCHUNK // 6, six, 0)
        scat_wait((NCHUNK - 1) % 2, (NCHUNK - 1) % 3)

    zero_acc()
    plsc.subcore_barrier()
    run(XWm, True)
    plsc.subcore_barrier()

    @pl.when(c == 0)
    def _():
        copy_out(Tout.at[0])

    @pl.when(c == 1)
    def _():
        copy_out(Tout.at[1])

    zero_acc()
    plsc.subcore_barrier()
    run(xwn, False)
    plsc.subcore_barrier()

    @pl.when(c == 0)
    def _():
        copy_out(Sout.at[0])

    @pl.when(c == 1)
    def _():
        copy_out(Sout.at[1])


_edge_call = pl.kernel(
    _edge_body,
    out_type=[jax.ShapeDtypeStruct((2, NP, H), jnp.float32),
              jax.ShapeDtypeStruct((2, NP, H), jnp.float32)],
    mesh=_mesh,
    scratch_types=[
        pltpu.VMEM((3, KE), jnp.int32),
        pltpu.VMEM((3, KE), jnp.int32),
        pltpu.VMEM((2, KE, H), jnp.float32),
        pltpu.VMEM((KE, H), jnp.float32),
        pltpu.VMEM_SHARED((NP, H), jnp.float32),
        pltpu.SemaphoreType.DMA,
        pltpu.SemaphoreType.DMA,
        pltpu.SemaphoreType.DMA,
        pltpu.SemaphoreType.DMA,
        pltpu.SemaphoreType.DMA,
        pltpu.SemaphoreType.DMA,
        pltpu.SemaphoreType.DMA,
        pltpu.SemaphoreType.DMA,
    ],
)


R = 1000
GRID = N // R
_f32 = jnp.float32


def _dotT(a, b):
    return lax.dot_general(a, b, (((1,), (1,)), ((), ())),
                           preferred_element_type=_f32)


def _dot(a, b):
    return lax.dot_general(a, b, (((1,), (0,)), ((), ())),
                           preferred_element_type=_f32)


def _dinv_of(degt):
    return lax.rsqrt(1.0 + degt[:, 0:1] + degt[:, 1:2])


def _emit_tables(X, dinv, Wm, Wc, Wr, br,
                 xw_ref, res_ref, XWm_ref, xwn_ref):
    XWm = _dot(X, Wm[...])
    xw = _dotT(X, Wc[...])
    xw_ref[...] = xw
    res_ref[...] = -(_dotT(xw, Wr[...]) + br[...])
    XWm_ref[...] = XWm
    xwn_ref[...] = xw * dinv


def _node_update(tp_ref, sp_ref, X, xw, res, dinv, bc, lam):
    T = tp_ref[0] + tp_ref[1]
    S = sp_ref[0] + sp_ref[1]
    conv = dinv * S + (dinv * dinv) * xw + bc[...]
    return jnp.maximum(conv + res, 0.0) + lam * (X * T)


def _pre_body(x_ref, degt_ref, We, be, Wm, Wc, Wr, br,
              X0_ref, xw_ref, res_ref, XWm_ref, xwn_ref):
    X0 = jnp.maximum(_dotT(x_ref[...], We[...]) + be[...], 0.0)
    X0_ref[...] = X0
    dinv = _dinv_of(degt_ref[...])
    _emit_tables(X0, dinv, Wm, Wc, Wr, br, xw_ref, res_ref, XWm_ref, xwn_ref)


def _mid_body(tp_ref, sp_ref, X_ref, xwin_ref, resin_ref, degt_ref,
              Wm, Wc, Wr, br, bc, lam_ref,
              X1_ref, xw_ref, res_ref, XWm_ref, xwn_ref):
    dinv = _dinv_of(degt_ref[...])
    X1 = _node_update(tp_ref, sp_ref, X_ref[...], xwin_ref[...],
                      resin_ref[...], dinv, bc, lam_ref[0, 0])
    X1_ref[...] = X1
    _emit_tables(X1, dinv, Wm, Wc, Wr, br, xw_ref, res_ref, XWm_ref, xwn_ref)


def _post_body(tp_ref, sp_ref, X_ref, xwin_ref, resin_ref, degt_ref,
               Wd, bd, bc, lam_ref, out_ref):
    dinv = _dinv_of(degt_ref[...])
    X2 = _node_update(tp_ref, sp_ref, X_ref[...], xwin_ref[...],
                      resin_ref[...], dinv, bc, lam_ref[0, 0])
    out_ref[...] = _dotT(X2, Wd[...]) + bd[...]


def _row_spec(w):
    return pl.BlockSpec((R, w), lambda i: (i, 0))


def _full_spec(shape):
    return pl.BlockSpec(shape, lambda i: tuple(0 for _ in shape))


_degt_spec = pl.BlockSpec((R, 2), lambda i: (i, 0))
_acc_spec = pl.BlockSpec((2, R, H), lambda i: (0, i, 0))
_lam_spec = pl.BlockSpec((1, 1), lambda i: (0, 0), memory_space=pltpu.SMEM)

_table_out_shapes = [
    jax.ShapeDtypeStruct((N, H), _f32),
    jax.ShapeDtypeStruct((N, H), _f32),
    jax.ShapeDtypeStruct((NP, H), _f32),
    jax.ShapeDtypeStruct((NP, H), _f32),
]
_table_out_specs = [_row_spec(H), _row_spec(H), _row_spec(H), _row_spec(H)]

_pre = pl.pallas_call(
    _pre_body,
    grid=(GRID,),
    in_specs=[_row_spec(H), _degt_spec, _full_spec((H, H)), _full_spec((1, H)),
              _full_spec((H, H)), _full_spec((H, H)), _full_spec((H, H)),
              _full_spec((1, H))],
    out_specs=[_row_spec(H)] + _table_out_specs,
    out_shape=[jax.ShapeDtypeStruct((N, H), _f32)] + _table_out_shapes,
)

_mid = pl.pallas_call(
    _mid_body,
    grid=(GRID,),
    in_specs=[_acc_spec, _acc_spec, _row_spec(H), _row_spec(H), _row_spec(H),
              _degt_spec, _full_spec((H, H)), _full_spec((H, H)),
              _full_spec((H, H)), _full_spec((1, H)), _full_spec((1, H)),
              _lam_spec],
    out_specs=[_row_spec(H)] + _table_out_specs,
    out_shape=[jax.ShapeDtypeStruct((N, H), _f32)] + _table_out_shapes,
)

_post = pl.pallas_call(
    _post_body,
    grid=(GRID,),
    in_specs=[_acc_spec, _acc_spec, _row_spec(H), _row_spec(H), _row_spec(H),
              _degt_spec, _full_spec((NCLASS, H)), _full_spec((1, NCLASS)),
              _full_spec((1, H)), _lam_spec],
    out_specs=[_row_spec(NCLASS)],
    out_shape=[jax.ShapeDtypeStruct((N, NCLASS), _f32)],
)


def kernel(x, edge_index, W_enc, b_enc, W_conv, b_conv, W_res, b_res,
           W_dec, b_dec, weight_mlp, lamda1):
    row = edge_index[0].astype(jnp.int32)
    col = edge_index[1].astype(jnp.int32)
    rowp = jnp.concatenate([row, jnp.arange(PAD, dtype=jnp.int32) % N])
    colp = jnp.concatenate([col, N + (jnp.arange(PAD, dtype=jnp.int32) % 16)])
    zrows = jnp.zeros((NP, H), _f32)

    be = b_enc.reshape(1, H)
    br = b_res.reshape(1, H)
    bc = b_conv.reshape(1, H)
    bd = b_dec.reshape(1, NCLASS)
    lam = lamda1.reshape(1, 1)

    degt = _deg_call(colp).T

    X0, xw1, res1, XWm1, xwn1 = _pre(x, degt, W_enc, be, weight_mlp,
                                     W_conv, W_res, br)
    tp1, sp1 = _edge_call(rowp, colp, XWm1, xwn1, zrows)
    X1, xw2, res2, XWm2, xwn2 = _mid(tp1, sp1, X0, xw1, res1, degt,
                                     weight_mlp, W_conv, W_res, br, bc, lam)
    tp2, sp2 = _edge_call(rowp, colp, XWm2, xwn2, zrows)
    (out,) = _post(tp2, sp2, X1, xw2, res2, degt, W_dec, bd, bc, lam)
    return out


# speedup vs baseline: 1.3418x; 1.0897x over previous
import functools

import jax
import jax.numpy as jnp
from jax import lax
from jax.experimental import pallas as pl
from jax.experimental.pallas import tpu as pltpu
from jax.experimental.pallas import tpu_sc as plsc

N = 10000
E = 320000
H = 128
NCLASS = 40
NP = 10112
ROWS_PER_TILE = NP // 16
NPD = 10240
DEG_ROWS_PER_TILE = NPD // 16
EP = 323584
PAD = EP - E
KE = 128
HALF_E = EP // 2
PPT = HALF_E // 16
NCHUNK = PPT // KE
KD = 128
DPT = HALF_E // 16
DCHUNK = DPT // KD

_mesh = plsc.VectorSubcoreMesh(core_axis_name="c", subcore_axis_name="s")


def _deg_body(colp, out, cbuf, ones, zv, degS, sem):
    c = lax.axis_index("c")
    s = lax.axis_index("s")
    for j in range(8):
        ones[pl.ds(16 * j, 16)] = jnp.ones((16,), jnp.float32)

    def zb(i, cy):
        zv[pl.ds(i * 16, 16)] = jnp.zeros((16,), jnp.float32)
        return cy

    lax.fori_loop(0, DEG_ROWS_PER_TILE // 16, zb, 0)
    pltpu.sync_copy(zv, degS.at[pl.ds(s * DEG_ROWS_PER_TILE, DEG_ROWS_PER_TILE)])
    plsc.subcore_barrier()

    def chunk(g, cy):
        base = c * HALF_E + s * DPT + g * KD
        pltpu.sync_copy(colp.at[pl.ds(base, KD)], cbuf.at[0])
        pltpu.sync_copy(ones, degS.at[cbuf.at[0]], add=True)
        return cy

    lax.fori_loop(0, DCHUNK, chunk, 0)
    plsc.subcore_barrier()

    @pl.when(c == 0)
    def _():
        pltpu.sync_copy(degS.at[pl.ds(s * DEG_ROWS_PER_TILE, DEG_ROWS_PER_TILE)],
                        out.at[0].at[pl.ds(s * DEG_ROWS_PER_TILE, DEG_ROWS_PER_TILE)])

    @pl.when(c == 1)
    def _():
        pltpu.sync_copy(degS.at[pl.ds(s * DEG_ROWS_PER_TILE, DEG_ROWS_PER_TILE)],
                        out.at[1].at[pl.ds(s * DEG_ROWS_PER_TILE, DEG_ROWS_PER_TILE)])


_deg_call = pl.kernel(
    _deg_body,
    out_type=jax.ShapeDtypeStruct((2, NPD), jnp.float32),
    mesh=_mesh,
    scratch_types=[
        pltpu.VMEM((1, KD), jnp.int32),
        pltpu.VMEM((KD,), jnp.float32),
        pltpu.VMEM((DEG_ROWS_PER_TILE,), jnp.float32),
        pltpu.VMEM_SHARED((NPD,), jnp.float32),
        pltpu.SemaphoreType.DMA,
    ],
)


def _edge_body(rowp, colp, XWm, xwn, zrows, Tout, Sout,
               rbuf, cbuf, sbuf, dbuf, accS,
               sem_s0, sem_s1, sem_d, sem_i0, sem_i1, sem_i2,
               sem_c0, sem_c1):
    c = lax.axis_index("c")
    s = lax.axis_index("s")
    sem_s = (sem_s0, sem_s1)
    sem_i = (sem_i0, sem_i1, sem_i2)
    sem_c = (sem_c0, sem_c1)

    def zero_acc():
        pltpu.sync_copy(zrows.at[pl.ds(s * ROWS_PER_TILE, ROWS_PER_TILE)],
                        accS.at[pl.ds(s * ROWS_PER_TILE, ROWS_PER_TILE)])

    def copy_out(dst):
        pltpu.sync_copy(accS.at[pl.ds(s * ROWS_PER_TILE, ROWS_PER_TILE)],
                        dst.at[pl.ds(s * ROWS_PER_TILE, ROWS_PER_TILE)])

    def run(table, with_diff):
        base0 = c * HALF_E + s * PPT

        def idxload(g, q):
            base = base0 + g * KE
            pltpu.async_copy(rowp.at[pl.ds(base, KE)], rbuf.at[q], sem_i[q])
            pltpu.async_copy(colp.at[pl.ds(base, KE)], cbuf.at[q], sem_i[q])

        def idxwait(g, q):
            base = base0 + g * KE
            pltpu.make_async_copy(rowp.at[pl.ds(base, KE)], rbuf.at[q],
                                  sem_i[q]).wait()
            pltpu.make_async_copy(colp.at[pl.ds(base, KE)], cbuf.at[q],
                                  sem_i[q]).wait()

        def rowg(q, p):
            pltpu.async_copy(table.at[rbuf.at[q]], sbuf.at[p], sem_s[p])

        def colg(q):
            pltpu.async_copy(XWm.at[cbuf.at[q]], dbuf, sem_d)

        def scat_wait(p, q):
            pltpu.make_async_copy(sbuf.at[p], accS.at[cbuf.at[q]],
                                  sem_c[p]).wait()

        def consume(g, p, q):
            pltpu.make_async_copy(table.at[rbuf.at[q]], sbuf.at[p],
                                  sem_s[p]).wait()
            if with_diff:
                pltpu.make_async_copy(XWm.at[cbuf.at[q]], dbuf, sem_d).wait()

                @plsc.parallel_loop(0, KE, 1, unroll=4)
                def _(i):
                    for j in range(8):
                        sl = pl.ds(16 * j, 16)
                        sbuf[p, i, sl] = jnp.maximum(
                            sbuf[p, i, sl] - dbuf[i, sl], 0.0)

                if isinstance(g, int):
                    if g + 1 < NCHUNK:
                        colg((g + 1) % 3)
                else:
                    @pl.when(g + 1 < NCHUNK)
                    def _():
                        colg((g + 1) % 3)

            pltpu.async_copy(sbuf.at[p], accS.at[cbuf.at[q]], sem_c[p],
                             add=True)

        idxload(0, 0)
        idxload(1, 1)
        idxwait(0, 0)
        rowg(0, 0)
        if with_diff:
            colg(0)

        def step(g, p, q):
            if isinstance(g, int):
                if g >= 1:
                    scat_wait(1 - p, (q + 2) % 3)
                if g + 2 < NCHUNK:
                    idxload(g + 2, (q + 2) % 3)
                if g + 1 < NCHUNK:
                    idxwait(g + 1, (q + 1) % 3)
                    rowg((q + 1) % 3, 1 - p)
            else:
                @pl.when(g >= 1)
                def _():
                    scat_wait(1 - p, (q + 2) % 3)

                @pl.when(g + 2 < NCHUNK)
                def _():
                    idxload(g + 2, (q + 2) % 3)

                @pl.when(g + 1 < NCHUNK)
                def _():
                    idxwait(g + 1, (q + 1) % 3)
                    rowg((q + 1) % 3, 1 - p)

            consume(g, p, q)

        def six(ii, cy):
            g0 = 6 * ii
            for u in range(6):
                step(g0 + u, u % 2, u % 3)
            return cy

        lax.fori_loop(0, NCHUNK // 6, six, 0)
        for g in range(6 * (NCHUNK // 6), NCHUNK):
            step(g, g % 2, g % 3)
        scat_wait((NCHUNK - 1) % 2, (NCHUNK - 1) % 3)

    zero_acc()
    plsc.subcore_barrier()
    run(XWm, True)
    plsc.subcore_barrier()

    @pl.when(c == 0)
    def _():
        copy_out(Tout.at[0])

    @pl.when(c == 1)
    def _():
        copy_out(Tout.at[1])

    zero_acc()
    plsc.subcore_barrier()
    run(xwn, False)
    plsc.subcore_barrier()

    @pl.when(c == 0)
    def _():
        copy_out(Sout.at[0])

    @pl.when(c == 1)
    def _():
        copy_out(Sout.at[1])


_edge_call = pl.kernel(
    _edge_body,
    out_type=[jax.ShapeDtypeStruct((2, NP, H), jnp.float32),
              jax.ShapeDtypeStruct((2, NP, H), jnp.float32)],
    mesh=_mesh,
    scratch_types=[
        pltpu.VMEM((3, KE), jnp.int32),
        pltpu.VMEM((3, KE), jnp.int32),
        pltpu.VMEM((2, KE, H), jnp.float32),
        pltpu.VMEM((KE, H), jnp.float32),
        pltpu.VMEM_SHARED((NP, H), jnp.float32),
        pltpu.SemaphoreType.DMA,
        pltpu.SemaphoreType.DMA,
        pltpu.SemaphoreType.DMA,
        pltpu.SemaphoreType.DMA,
        pltpu.SemaphoreType.DMA,
        pltpu.SemaphoreType.DMA,
        pltpu.SemaphoreType.DMA,
        pltpu.SemaphoreType.DMA,
    ],
)


R = 1000
GRID = N // R
_f32 = jnp.float32


def _dotT(a, b):
    return lax.dot_general(a, b, (((1,), (1,)), ((), ())),
                           preferred_element_type=_f32)


def _dot(a, b):
    return lax.dot_general(a, b, (((1,), (0,)), ((), ())),
                           preferred_element_type=_f32)


def _dinv_of(degt):
    return lax.rsqrt(1.0 + degt[:, 0:1] + degt[:, 1:2])


def _emit_tables(X, dinv, Wm, Wc, Wr, br,
                 xw_ref, res_ref, XWm_ref, xwn_ref):
    XWm = _dot(X, Wm[...])
    xw = _dotT(X, Wc[...])
    xw_ref[...] = xw
    res_ref[...] = -(_dotT(xw, Wr[...]) + br[...])
    XWm_ref[...] = XWm
    xwn_ref[...] = xw * dinv


def _node_update(tp_ref, sp_ref, X, xw, res, dinv, bc, lam):
    T = tp_ref[0] + tp_ref[1]
    S = sp_ref[0] + sp_ref[1]
    conv = dinv * S + (dinv * dinv) * xw + bc[...]
    return jnp.maximum(conv + res, 0.0) + lam * (X * T)


def _pre_body(x_ref, degt_ref, We, be, Wm, Wc, Wr, br,
              X0_ref, xw_ref, res_ref, XWm_ref, xwn_ref):
    X0 = jnp.maximum(_dotT(x_ref[...], We[...]) + be[...], 0.0)
    X0_ref[...] = X0
    dinv = _dinv_of(degt_ref[...])
    _emit_tables(X0, dinv, Wm, Wc, Wr, br, xw_ref, res_ref, XWm_ref, xwn_ref)


def _mid_body(tp_ref, sp_ref, X_ref, xwin_ref, resin_ref, degt_ref,
              Wm, Wc, Wr, br, bc, lam_ref,
              X1_ref, xw_ref, res_ref, XWm_ref, xwn_ref):
    dinv = _dinv_of(degt_ref[...])
    X1 = _node_update(tp_ref, sp_ref, X_ref[...], xwin_ref[...],
                      resin_ref[...], dinv, bc, lam_ref[0, 0])
    X1_ref[...] = X1
    _emit_tables(X1, dinv, Wm, Wc, Wr, br, xw_ref, res_ref, XWm_ref, xwn_ref)


def _post_body(tp_ref, sp_ref, X_ref, xwin_ref, resin_ref, degt_ref,
               Wd, bd, bc, lam_ref, out_ref):
    dinv = _dinv_of(degt_ref[...])
    X2 = _node_update(tp_ref, sp_ref, X_ref[...], xwin_ref[...],
                      resin_ref[...], dinv, bc, lam_ref[0, 0])
    out_ref[...] = _dotT(X2, Wd[...]) + bd[...]


def _row_spec(w):
    return pl.BlockSpec((R, w), lambda i: (i, 0))


def _full_spec(shape):
    return pl.BlockSpec(shape, lambda i: tuple(0 for _ in shape))


_degt_spec = pl.BlockSpec((R, 2), lambda i: (i, 0))
_acc_spec = pl.BlockSpec((2, R, H), lambda i: (0, i, 0))
_lam_spec = pl.BlockSpec((1, 1), lambda i: (0, 0), memory_space=pltpu.SMEM)

_table_out_shapes = [
    jax.ShapeDtypeStruct((N, H), _f32),
    jax.ShapeDtypeStruct((N, H), _f32),
    jax.ShapeDtypeStruct((NP, H), _f32),
    jax.ShapeDtypeStruct((NP, H), _f32),
]
_table_out_specs = [_row_spec(H), _row_spec(H), _row_spec(H), _row_spec(H)]

_pre = pl.pallas_call(
    _pre_body,
    grid=(GRID,),
    in_specs=[_row_spec(H), _degt_spec, _full_spec((H, H)), _full_spec((1, H)),
              _full_spec((H, H)), _full_spec((H, H)), _full_spec((H, H)),
              _full_spec((1, H))],
    out_specs=[_row_spec(H)] + _table_out_specs,
    out_shape=[jax.ShapeDtypeStruct((N, H), _f32)] + _table_out_shapes,
)

_mid = pl.pallas_call(
    _mid_body,
    grid=(GRID,),
    in_specs=[_acc_spec, _acc_spec, _row_spec(H), _row_spec(H), _row_spec(H),
              _degt_spec, _full_spec((H, H)), _full_spec((H, H)),
              _full_spec((H, H)), _full_spec((1, H)), _full_spec((1, H)),
              _lam_spec],
    out_specs=[_row_spec(H)] + _table_out_specs,
    out_shape=[jax.ShapeDtypeStruct((N, H), _f32)] + _table_out_shapes,
)

_post = pl.pallas_call(
    _post_body,
    grid=(GRID,),
    in_specs=[_acc_spec, _acc_spec, _row_spec(H), _row_spec(H), _row_spec(H),
              _degt_spec, _full_spec((NCLASS, H)), _full_spec((1, NCLASS)),
              _full_spec((1, H)), _lam_spec],
    out_specs=[_row_spec(NCLASS)],
    out_shape=[jax.ShapeDtypeStruct((N, NCLASS), _f32)],
)


def kernel(x, edge_index, W_enc, b_enc, W_conv, b_conv, W_res, b_res,
           W_dec, b_dec, weight_mlp, lamda1):
    row = edge_index[0].astype(jnp.int32)
    col = edge_index[1].astype(jnp.int32)
    rowp = jnp.concatenate([row, jnp.arange(PAD, dtype=jnp.int32) % N])
    colp = jnp.concatenate([col, N + (jnp.arange(PAD, dtype=jnp.int32) % 16)])
    zrows = jnp.zeros((NP, H), _f32)

    be = b_enc.reshape(1, H)
    br = b_res.reshape(1, H)
    bc = b_conv.reshape(1, H)
    bd = b_dec.reshape(1, NCLASS)
    lam = lamda1.reshape(1, 1)

    degt = _deg_call(colp).T

    X0, xw1, res1, XWm1, xwn1 = _pre(x, degt, W_enc, be, weight_mlp,
                                     W_conv, W_res, br)
    tp1, sp1 = _edge_call(rowp, colp, XWm1, xwn1, zrows)
    X1, xw2, res2, XWm2, xwn2 = _mid(tp1, sp1, X0, xw1, res1, degt,
                                     weight_mlp, W_conv, W_res, br, bc, lam)
    tp2, sp2 = _edge_call(rowp, colp, XWm2, xwn2, zrows)
    (out,) = _post(tp2, sp2, X1, xw2, res2, degt, W_dec, bd, bc, lam)
    return out


# speedup vs baseline: 1.3531x; 1.0084x over previous
import functools

import jax
import jax.numpy as jnp
from jax import lax
from jax.experimental import pallas as pl
from jax.experimental.pallas import tpu as pltpu
from jax.experimental.pallas import tpu_sc as plsc

N = 10000
E = 320000
H = 128
NCLASS = 40
NP = 10112
ROWS_PER_TILE = NP // 16
NPD = 10240
DEG_ROWS_PER_TILE = NPD // 16
EP = 323584
PAD = EP - E
KE = 128
HALF_E = EP // 2
PPT = HALF_E // 16
NCHUNK = PPT // KE
KD = 128
DPT = HALF_E // 16
DCHUNK = DPT // KD

_mesh = plsc.VectorSubcoreMesh(core_axis_name="c", subcore_axis_name="s")


def _deg_body(colp, out, cbuf, ones, zv, degS, sem):
    c = lax.axis_index("c")
    s = lax.axis_index("s")
    for j in range(8):
        ones[pl.ds(16 * j, 16)] = jnp.ones((16,), jnp.float32)

    def zb(i, cy):
        zv[pl.ds(i * 16, 16)] = jnp.zeros((16,), jnp.float32)
        return cy

    lax.fori_loop(0, DEG_ROWS_PER_TILE // 16, zb, 0)
    pltpu.sync_copy(zv, degS.at[pl.ds(s * DEG_ROWS_PER_TILE, DEG_ROWS_PER_TILE)])
    plsc.subcore_barrier()

    def chunk(g, cy):
        base = c * HALF_E + s * DPT + g * KD
        pltpu.sync_copy(colp.at[pl.ds(base, KD)], cbuf.at[0])
        pltpu.sync_copy(ones, degS.at[cbuf.at[0]], add=True)
        return cy

    lax.fori_loop(0, DCHUNK, chunk, 0)
    plsc.subcore_barrier()

    @pl.when(c == 0)
    def _():
        pltpu.sync_copy(degS.at[pl.ds(s * DEG_ROWS_PER_TILE, DEG_ROWS_PER_TILE)],
                        out.at[0].at[pl.ds(s * DEG_ROWS_PER_TILE, DEG_ROWS_PER_TILE)])

    @pl.when(c == 1)
    def _():
        pltpu.sync_copy(degS.at[pl.ds(s * DEG_ROWS_PER_TILE, DEG_ROWS_PER_TILE)],
                        out.at[1].at[pl.ds(s * DEG_ROWS_PER_TILE, DEG_ROWS_PER_TILE)])


_deg_call = pl.kernel(
    _deg_body,
    out_type=jax.ShapeDtypeStruct((2, NPD), jnp.float32),
    mesh=_mesh,
    scratch_types=[
        pltpu.VMEM((1, KD), jnp.int32),
        pltpu.VMEM((KD,), jnp.float32),
        pltpu.VMEM((DEG_ROWS_PER_TILE,), jnp.float32),
        pltpu.VMEM_SHARED((NPD,), jnp.float32),
        pltpu.SemaphoreType.DMA,
    ],
)


def _edge_body(rowp, colp, XWm, xwn, zrows, Tout, Sout,
               rbuf, cbuf, sbuf, dbuf, accS,
               sem_s0, sem_s1, sem_d, sem_i0, sem_i1, sem_i2,
               sem_c0, sem_c1):
    c = lax.axis_index("c")
    s = lax.axis_index("s")
    sem_s = (sem_s0, sem_s1)
    sem_i = (sem_i0, sem_i1, sem_i2)
    sem_c = (sem_c0, sem_c1)

    def zero_acc():
        pltpu.sync_copy(zrows.at[pl.ds(s * ROWS_PER_TILE, ROWS_PER_TILE)],
                        accS.at[pl.ds(s * ROWS_PER_TILE, ROWS_PER_TILE)])

    def copy_out(dst):
        pltpu.sync_copy(accS.at[pl.ds(s * ROWS_PER_TILE, ROWS_PER_TILE)],
                        dst.at[pl.ds(s * ROWS_PER_TILE, ROWS_PER_TILE)])

    def run(table, with_diff):
        base0 = c * HALF_E + s * PPT

        def idxload(g, q):
            base = base0 + g * KE
            pltpu.async_copy(rowp.at[pl.ds(base, KE)], rbuf.at[q], sem_i[q])
            pltpu.async_copy(colp.at[pl.ds(base, KE)], cbuf.at[q], sem_i[q])

        def idxwait(g, q):
            base = base0 + g * KE
            pltpu.make_async_copy(rowp.at[pl.ds(base, KE)], rbuf.at[q],
                                  sem_i[q]).wait()
            pltpu.make_async_copy(colp.at[pl.ds(base, KE)], cbuf.at[q],
                                  sem_i[q]).wait()

        def rowg(q, p):
            pltpu.async_copy(table.at[rbuf.at[q]], sbuf.at[p], sem_s[p])

        def colg(q):
            pltpu.async_copy(XWm.at[cbuf.at[q]], dbuf, sem_d)

        def scat_wait(p, q):
            pltpu.make_async_copy(sbuf.at[p], accS.at[cbuf.at[q]],
                                  sem_c[p]).wait()

        def consume(g, p, q):
            pltpu.make_async_copy(table.at[rbuf.at[q]], sbuf.at[p],
                                  sem_s[p]).wait()
            if with_diff:
                pltpu.make_async_copy(XWm.at[cbuf.at[q]], dbuf, sem_d).wait()

                @plsc.parallel_loop(0, KE, 1, unroll=4)
                def _(i):
                    for j in range(8):
                        sl = pl.ds(16 * j, 16)
                        sbuf[p, i, sl] = jnp.maximum(
                            sbuf[p, i, sl] - dbuf[i, sl], 0.0)

                if isinstance(g, int):
                    if g + 1 < NCHUNK:
                        colg((g + 1) % 3)
                else:
                    @pl.when(g + 1 < NCHUNK)
                    def _():
                        colg((g + 1) % 3)

            pltpu.async_copy(sbuf.at[p], accS.at[cbuf.at[q]], sem_c[p],
                             add=True)

        idxload(0, 0)
        idxload(1, 1)
        idxwait(0, 0)
        rowg(0, 0)
        if with_diff:
            colg(0)

        def step(g, p, q):
            if isinstance(g, int):
                if g >= 1:
                    scat_wait(1 - p, (q + 2) % 3)
                if g + 2 < NCHUNK:
                    idxload(g + 2, (q + 2) % 3)
                if g + 1 < NCHUNK:
                    idxwait(g + 1, (q + 1) % 3)
                    rowg((q + 1) % 3, 1 - p)
            else:
                @pl.when(g >= 1)
                def _():
                    scat_wait(1 - p, (q + 2) % 3)

                @pl.when(g + 2 < NCHUNK)
                def _():
                    idxload(g + 2, (q + 2) % 3)

                @pl.when(g + 1 < NCHUNK)
                def _():
                    idxwait(g + 1, (q + 1) % 3)
                    rowg((q + 1) % 3, 1 - p)

            consume(g, p, q)

        def six(ii, cy):
            g0 = 6 * ii
            for u in range(6):
                step(g0 + u, u % 2, u % 3)
            return cy

        lax.fori_loop(0, NCHUNK // 6, six, 0)
        for g in range(6 * (NCHUNK // 6), NCHUNK):
            step(g, g % 2, g % 3)
        scat_wait((NCHUNK - 1) % 2, (NCHUNK - 1) % 3)

    zero_acc()
    plsc.subcore_barrier()
    run(XWm, True)
    plsc.subcore_barrier()

    @pl.when(c == 0)
    def _():
        copy_out(Tout.at[0])

    @pl.when(c == 1)
    def _():
        copy_out(Tout.at[1])

    zero_acc()
    plsc.subcore_barrier()
    run(xwn, False)
    plsc.subcore_barrier()

    @pl.when(c == 0)
    def _():
        copy_out(Sout.at[0])

    @pl.when(c == 1)
    def _():
        copy_out(Sout.at[1])


_edge_call = pl.kernel(
    _edge_body,
    out_type=[jax.ShapeDtypeStruct((2, NP, H), jnp.float32),
              jax.ShapeDtypeStruct((2, NP, H), jnp.float32)],
    mesh=_mesh,
    scratch_types=[
        pltpu.VMEM((3, KE), jnp.int32),
        pltpu.VMEM((3, KE), jnp.int32),
        pltpu.VMEM((2, KE, H), jnp.float32),
        pltpu.VMEM((KE, H), jnp.float32),
        pltpu.VMEM_SHARED((NP, H), jnp.float32),
        pltpu.SemaphoreType.DMA,
        pltpu.SemaphoreType.DMA,
        pltpu.SemaphoreType.DMA,
        pltpu.SemaphoreType.DMA,
        pltpu.SemaphoreType.DMA,
        pltpu.SemaphoreType.DMA,
        pltpu.SemaphoreType.DMA,
        pltpu.SemaphoreType.DMA,
    ],
)


R = 2000
GRID = N // R
_f32 = jnp.float32


def _dotT(a, b):
    return lax.dot_general(a, b, (((1,), (1,)), ((), ())),
                           preferred_element_type=_f32)


def _dot(a, b):
    return lax.dot_general(a, b, (((1,), (0,)), ((), ())),
                           preferred_element_type=_f32)


def _dinv_of(degt):
    return lax.rsqrt(1.0 + degt[:, 0:1] + degt[:, 1:2])


def _emit_tables(X, dinv, Wm, Wc, Wr, br,
                 xw_ref, res_ref, XWm_ref, xwn_ref):
    XWm = _dot(X, Wm[...])
    xw = _dotT(X, Wc[...])
    xw_ref[...] = xw
    res_ref[...] = -(_dotT(xw, Wr[...]) + br[...])
    XWm_ref[...] = XWm
    xwn_ref[...] = xw * dinv


def _node_update(tp_ref, sp_ref, X, xw, res, dinv, bc, lam):
    T = tp_ref[0] + tp_ref[1]
    S = sp_ref[0] + sp_ref[1]
    conv = dinv * S + (dinv * dinv) * xw + bc[...]
    return jnp.maximum(conv + res, 0.0) + lam * (X * T)


def _pre_body(x_ref, degt_ref, We, be, Wm, Wc, Wr, br,
              X0_ref, xw_ref, res_ref, XWm_ref, xwn_ref):
    X0 = jnp.maximum(_dotT(x_ref[...], We[...]) + be[...], 0.0)
    X0_ref[...] = X0
    dinv = _dinv_of(degt_ref[...])
    _emit_tables(X0, dinv, Wm, Wc, Wr, br, xw_ref, res_ref, XWm_ref, xwn_ref)


def _mid_body(tp_ref, sp_ref, X_ref, xwin_ref, resin_ref, degt_ref,
              Wm, Wc, Wr, br, bc, lam_ref,
              X1_ref, xw_ref, res_ref, XWm_ref, xwn_ref):
    dinv = _dinv_of(degt_ref[...])
    X1 = _node_update(tp_ref, sp_ref, X_ref[...], xwin_ref[...],
                      resin_ref[...], dinv, bc, lam_ref[0, 0])
    X1_ref[...] = X1
    _emit_tables(X1, dinv, Wm, Wc, Wr, br, xw_ref, res_ref, XWm_ref, xwn_ref)


def _post_body(tp_ref, sp_ref, X_ref, xwin_ref, resin_ref, degt_ref,
               Wd, bd, bc, lam_ref, out_ref):
    dinv = _dinv_of(degt_ref[...])
    X2 = _node_update(tp_ref, sp_ref, X_ref[...], xwin_ref[...],
                      resin_ref[...], dinv, bc, lam_ref[0, 0])
    out_ref[...] = _dotT(X2, Wd[...]) + bd[...]


def _row_spec(w):
    return pl.BlockSpec((R, w), lambda i: (i, 0))


def _full_spec(shape):
    return pl.BlockSpec(shape, lambda i: tuple(0 for _ in shape))


_degt_spec = pl.BlockSpec((R, 2), lambda i: (i, 0))
_acc_spec = pl.BlockSpec((2, R, H), lambda i: (0, i, 0))
_lam_spec = pl.BlockSpec((1, 1), lambda i: (0, 0), memory_space=pltpu.SMEM)

_table_out_shapes = [
    jax.ShapeDtypeStruct((N, H), _f32),
    jax.ShapeDtypeStruct((N, H), _f32),
    jax.ShapeDtypeStruct((NP, H), _f32),
    jax.ShapeDtypeStruct((NP, H), _f32),
]
_table_out_specs = [_row_spec(H), _row_spec(H), _row_spec(H), _row_spec(H)]

_pre = pl.pallas_call(
    _pre_body,
    grid=(GRID,),
    in_specs=[_row_spec(H), _degt_spec, _full_spec((H, H)), _full_spec((1, H)),
              _full_spec((H, H)), _full_spec((H, H)), _full_spec((H, H)),
              _full_spec((1, H))],
    out_specs=[_row_spec(H)] + _table_out_specs,
    out_shape=[jax.ShapeDtypeStruct((N, H), _f32)] + _table_out_shapes,
)

_mid = pl.pallas_call(
    _mid_body,
    grid=(GRID,),
    in_specs=[_acc_spec, _acc_spec, _row_spec(H), _row_spec(H), _row_spec(H),
              _degt_spec, _full_spec((H, H)), _full_spec((H, H)),
              _full_spec((H, H)), _full_spec((1, H)), _full_spec((1, H)),
              _lam_spec],
    out_specs=[_row_spec(H)] + _table_out_specs,
    out_shape=[jax.ShapeDtypeStruct((N, H), _f32)] + _table_out_shapes,
)

_post = pl.pallas_call(
    _post_body,
    grid=(GRID,),
    in_specs=[_acc_spec, _acc_spec, _row_spec(H), _row_spec(H), _row_spec(H),
              _degt_spec, _full_spec((NCLASS, H)), _full_spec((1, NCLASS)),
              _full_spec((1, H)), _lam_spec],
    out_specs=[_row_spec(NCLASS)],
    out_shape=[jax.ShapeDtypeStruct((N, NCLASS), _f32)],
)


def kernel(x, edge_index, W_enc, b_enc, W_conv, b_conv, W_res, b_res,
           W_dec, b_dec, weight_mlp, lamda1):
    row = edge_index[0].astype(jnp.int32)
    col = edge_index[1].astype(jnp.int32)
    rowp = jnp.concatenate([row, jnp.arange(PAD, dtype=jnp.int32) % N])
    colp = jnp.concatenate([col, N + (jnp.arange(PAD, dtype=jnp.int32) % 16)])
    zrows = jnp.zeros((NP, H), _f32)

    be = b_enc.reshape(1, H)
    br = b_res.reshape(1, H)
    bc = b_conv.reshape(1, H)
    bd = b_dec.reshape(1, NCLASS)
    lam = lamda1.reshape(1, 1)

    degt = _deg_call(colp).T

    X0, xw1, res1, XWm1, xwn1 = _pre(x, degt, W_enc, be, weight_mlp,
                                     W_conv, W_res, br)
    tp1, sp1 = _edge_call(rowp, colp, XWm1, xwn1, zrows)
    X1, xw2, res2, XWm2, xwn2 = _mid(tp1, sp1, X0, xw1, res1, degt,
                                     weight_mlp, W_conv, W_res, br, bc, lam)
    tp2, sp2 = _edge_call(rowp, colp, XWm2, xwn2, zrows)
    (out,) = _post(tp2, sp2, X1, xw2, res2, degt, W_dec, bd, bc, lam)
    return out


# speedup vs baseline: 1.4060x; 1.0391x over previous
import functools

import jax
import jax.numpy as jnp
from jax import lax
from jax.experimental import pallas as pl
from jax.experimental.pallas import tpu as pltpu
from jax.experimental.pallas import tpu_sc as plsc

N = 10000
E = 320000
H = 128
NCLASS = 40
NP = 10112
ROWS_PER_TILE = NP // 16
NPD = 10240
DEG_ROWS_PER_TILE = NPD // 16
EP = 323584
PAD = EP - E
KE = 128
HALF_E = EP // 2
PPT = HALF_E // 16
NCHUNK = PPT // KE
KD = 128
DPT = HALF_E // 16
DCHUNK = DPT // KD

_mesh = plsc.VectorSubcoreMesh(core_axis_name="c", subcore_axis_name="s")


def _deg_body(colp, out, cbuf, ones, zv, degS,
              sem_i0, sem_i1, sem_i2, sem_c0, sem_c1):
    c = lax.axis_index("c")
    s = lax.axis_index("s")
    sem_i = (sem_i0, sem_i1, sem_i2)
    sem_c = (sem_c0, sem_c1)
    for j in range(8):
        ones[pl.ds(16 * j, 16)] = jnp.ones((16,), jnp.float32)

    def zb(i, cy):
        zv[pl.ds(i * 16, 16)] = jnp.zeros((16,), jnp.float32)
        return cy

    lax.fori_loop(0, DEG_ROWS_PER_TILE // 16, zb, 0)
    pltpu.sync_copy(zv, degS.at[pl.ds(s * DEG_ROWS_PER_TILE, DEG_ROWS_PER_TILE)])
    plsc.subcore_barrier()

    def idxload(g, q):
        base = c * HALF_E + s * DPT + g * KD
        pltpu.async_copy(colp.at[pl.ds(base, KD)], cbuf.at[q], sem_i[q])

    def idxwait(g, q):
        base = c * HALF_E + s * DPT + g * KD
        pltpu.make_async_copy(colp.at[pl.ds(base, KD)], cbuf.at[q],
                              sem_i[q]).wait()

    def scat_wait(p, q):
        pltpu.make_async_copy(ones, degS.at[cbuf.at[q]], sem_c[p]).wait()

    def step(g, p, q):
        if isinstance(g, int):
            if g >= 1:
                scat_wait(1 - p, (q + 2) % 3)
            if g + 2 < DCHUNK:
                idxload(g + 2, (q + 2) % 3)
        else:
            @pl.when(g >= 1)
            def _():
                scat_wait(1 - p, (q + 2) % 3)

            @pl.when(g + 2 < DCHUNK)
            def _():
                idxload(g + 2, (q + 2) % 3)

        idxwait(g, q)
        pltpu.async_copy(ones, degS.at[cbuf.at[q]], sem_c[p], add=True)

    idxload(0, 0)
    idxload(1, 1)

    def six(ii, cy):
        g0 = 6 * ii
        for u in range(6):
            step(g0 + u, u % 2, u % 3)
        return cy

    lax.fori_loop(0, DCHUNK // 6, six, 0)
    for g in range(6 * (DCHUNK // 6), DCHUNK):
        step(g, g % 2, g % 3)
    scat_wait((DCHUNK - 1) % 2, (DCHUNK - 1) % 3)
    plsc.subcore_barrier()

    @pl.when(c == 0)
    def _():
        pltpu.sync_copy(degS.at[pl.ds(s * DEG_ROWS_PER_TILE, DEG_ROWS_PER_TILE)],
                        out.at[0].at[pl.ds(s * DEG_ROWS_PER_TILE, DEG_ROWS_PER_TILE)])

    @pl.when(c == 1)
    def _():
        pltpu.sync_copy(degS.at[pl.ds(s * DEG_ROWS_PER_TILE, DEG_ROWS_PER_TILE)],
                        out.at[1].at[pl.ds(s * DEG_ROWS_PER_TILE, DEG_ROWS_PER_TILE)])


_deg_call = pl.kernel(
    _deg_body,
    out_type=jax.ShapeDtypeStruct((2, NPD), jnp.float32),
    mesh=_mesh,
    scratch_types=[
        pltpu.VMEM((3, KD), jnp.int32),
        pltpu.VMEM((KD,), jnp.float32),
        pltpu.VMEM((DEG_ROWS_PER_TILE,), jnp.float32),
        pltpu.VMEM_SHARED((NPD,), jnp.float32),
        pltpu.SemaphoreType.DMA,
        pltpu.SemaphoreType.DMA,
        pltpu.SemaphoreType.DMA,
        pltpu.SemaphoreType.DMA,
        pltpu.SemaphoreType.DMA,
    ],
)


def _edge_body(rowp, colp, XWm, xwn, zrows, Tout, Sout,
               rbuf, cbuf, sbuf, dbuf, accS,
               sem_s0, sem_s1, sem_d, sem_i0, sem_i1, sem_i2,
               sem_c0, sem_c1):
    c = lax.axis_index("c")
    s = lax.axis_index("s")
    sem_s = (sem_s0, sem_s1)
    sem_i = (sem_i0, sem_i1, sem_i2)
    sem_c = (sem_c0, sem_c1)

    def zero_acc():
        pltpu.sync_copy(zrows.at[pl.ds(s * ROWS_PER_TILE, ROWS_PER_TILE)],
                        accS.at[pl.ds(s * ROWS_PER_TILE, ROWS_PER_TILE)])

    def copy_out(dst):
        pltpu.sync_copy(accS.at[pl.ds(s * ROWS_PER_TILE, ROWS_PER_TILE)],
                        dst.at[pl.ds(s * ROWS_PER_TILE, ROWS_PER_TILE)])

    def run(table, with_diff):
        base0 = c * HALF_E + s * PPT

        def idxload(g, q):
            base = base0 + g * KE
            pltpu.async_copy(rowp.at[pl.ds(base, KE)], rbuf.at[q], sem_i[q])
            pltpu.async_copy(colp.at[pl.ds(base, KE)], cbuf.at[q], sem_i[q])

        def idxwait(g, q):
            base = base0 + g * KE
            pltpu.make_async_copy(rowp.at[pl.ds(base, KE)], rbuf.at[q],
                                  sem_i[q]).wait()
            pltpu.make_async_copy(colp.at[pl.ds(base, KE)], cbuf.at[q],
                                  sem_i[q]).wait()

        def rowg(q, p):
            pltpu.async_copy(table.at[rbuf.at[q]], sbuf.at[p], sem_s[p])

        def colg(q):
            pltpu.async_copy(XWm.at[cbuf.at[q]], dbuf, sem_d)

        def scat_wait(p, q):
            pltpu.make_async_copy(sbuf.at[p], accS.at[cbuf.at[q]],
                                  sem_c[p]).wait()

        def consume(g, p, q):
            pltpu.make_async_copy(table.at[rbuf.at[q]], sbuf.at[p],
                                  sem_s[p]).wait()
            if with_diff:
                pltpu.make_async_copy(XWm.at[cbuf.at[q]], dbuf, sem_d).wait()

                @plsc.parallel_loop(0, KE, 1, unroll=4)
                def _(i):
                    for j in range(8):
                        sl = pl.ds(16 * j, 16)
                        sbuf[p, i, sl] = jnp.maximum(
                            sbuf[p, i, sl] - dbuf[i, sl], 0.0)

                if isinstance(g, int):
                    if g + 1 < NCHUNK:
                        colg((g + 1) % 3)
                else:
                    @pl.when(g + 1 < NCHUNK)
                    def _():
                        colg((g + 1) % 3)

            pltpu.async_copy(sbuf.at[p], accS.at[cbuf.at[q]], sem_c[p],
                             add=True)

        idxload(0, 0)
        idxload(1, 1)
        idxwait(0, 0)
        rowg(0, 0)
        if with_diff:
            colg(0)

        def step(g, p, q):
            if isinstance(g, int):
                if g >= 1:
                    scat_wait(1 - p, (q + 2) % 3)
                if g + 2 < NCHUNK:
                    idxload(g + 2, (q + 2) % 3)
                if g + 1 < NCHUNK:
                    idxwait(g + 1, (q + 1) % 3)
                    rowg((q + 1) % 3, 1 - p)
            else:
                @pl.when(g >= 1)
                def _():
                    scat_wait(1 - p, (q + 2) % 3)

                @pl.when(g + 2 < NCHUNK)
                def _():
                    idxload(g + 2, (q + 2) % 3)

                @pl.when(g + 1 < NCHUNK)
                def _():
                    idxwait(g + 1, (q + 1) % 3)
                    rowg((q + 1) % 3, 1 - p)

            consume(g, p, q)

        def six(ii, cy):
            g0 = 6 * ii
            for u in range(6):
                step(g0 + u, u % 2, u % 3)
            return cy

        lax.fori_loop(0, NCHUNK // 6, six, 0)
        for g in range(6 * (NCHUNK // 6), NCHUNK):
            step(g, g % 2, g % 3)
        scat_wait((NCHUNK - 1) % 2, (NCHUNK - 1) % 3)

    zero_acc()
    plsc.subcore_barrier()
    run(XWm, True)
    plsc.subcore_barrier()

    @pl.when(c == 0)
    def _():
        copy_out(Tout.at[0])

    @pl.when(c == 1)
    def _():
        copy_out(Tout.at[1])

    zero_acc()
    plsc.subcore_barrier()
    run(xwn, False)
    plsc.subcore_barrier()

    @pl.when(c == 0)
    def _():
        copy_out(Sout.at[0])

    @pl.when(c == 1)
    def _():
        copy_out(Sout.at[1])


_edge_call = pl.kernel(
    _edge_body,
    out_type=[jax.ShapeDtypeStruct((2, NP, H), jnp.float32),
              jax.ShapeDtypeStruct((2, NP, H), jnp.float32)],
    mesh=_mesh,
    scratch_types=[
        pltpu.VMEM((3, KE), jnp.int32),
        pltpu.VMEM((3, KE), jnp.int32),
        pltpu.VMEM((2, KE, H), jnp.float32),
        pltpu.VMEM((KE, H), jnp.float32),
        pltpu.VMEM_SHARED((NP, H), jnp.float32),
        pltpu.SemaphoreType.DMA,
        pltpu.SemaphoreType.DMA,
        pltpu.SemaphoreType.DMA,
        pltpu.SemaphoreType.DMA,
        pltpu.SemaphoreType.DMA,
        pltpu.SemaphoreType.DMA,
        pltpu.SemaphoreType.DMA,
        pltpu.SemaphoreType.DMA,
    ],
)


R = 2000
GRID = N // R
_f32 = jnp.float32


def _dotT(a, b):
    return lax.dot_general(a, b, (((1,), (1,)), ((), ())),
                           preferred_element_type=_f32)


def _dot(a, b):
    return lax.dot_general(a, b, (((1,), (0,)), ((), ())),
                           preferred_element_type=_f32)


def _dinv_of(degt):
    return lax.rsqrt(1.0 + degt[:, 0:1] + degt[:, 1:2])


def _emit_tables(X, dinv, Wm, Wc, Wr, br,
                 xw_ref, res_ref, XWm_ref, xwn_ref):
    XWm = _dot(X, Wm[...])
    xw = _dotT(X, Wc[...])
    xw_ref[...] = xw
    res_ref[...] = -(_dotT(xw, Wr[...]) + br[...])
    XWm_ref[...] = XWm
    xwn_ref[...] = xw * dinv


def _node_update(tp_ref, sp_ref, X, xw, res, dinv, bc, lam):
    T = tp_ref[0] + tp_ref[1]
    S = sp_ref[0] + sp_ref[1]
    conv = dinv * S + (dinv * dinv) * xw + bc[...]
    return jnp.maximum(conv + res, 0.0) + lam * (X * T)


def _pre_body(x_ref, degt_ref, We, be, Wm, Wc, Wr, br,
              X0_ref, xw_ref, res_ref, XWm_ref, xwn_ref):
    X0 = jnp.maximum(_dotT(x_ref[...], We[...]) + be[...], 0.0)
    X0_ref[...] = X0
    dinv = _dinv_of(degt_ref[...])
    _emit_tables(X0, dinv, Wm, Wc, Wr, br, xw_ref, res_ref, XWm_ref, xwn_ref)


def _mid_body(tp_ref, sp_ref, X_ref, xwin_ref, resin_ref, degt_ref,
              Wm, Wc, Wr, br, bc, lam_ref,
              X1_ref, xw_ref, res_ref, XWm_ref, xwn_ref):
    dinv = _dinv_of(degt_ref[...])
    X1 = _node_update(tp_ref, sp_ref, X_ref[...], xwin_ref[...],
                      resin_ref[...], dinv, bc, lam_ref[0, 0])
    X1_ref[...] = X1
    _emit_tables(X1, dinv, Wm, Wc, Wr, br, xw_ref, res_ref, XWm_ref, xwn_ref)


def _post_body(tp_ref, sp_ref, X_ref, xwin_ref, resin_ref, degt_ref,
               Wd, bd, bc, lam_ref, out_ref):
    dinv = _dinv_of(degt_ref[...])
    X2 = _node_update(tp_ref, sp_ref, X_ref[...], xwin_ref[...],
                      resin_ref[...], dinv, bc, lam_ref[0, 0])
    out_ref[...] = _dotT(X2, Wd[...]) + bd[...]


def _row_spec(w):
    return pl.BlockSpec((R, w), lambda i: (i, 0))


def _full_spec(shape):
    return pl.BlockSpec(shape, lambda i: tuple(0 for _ in shape))


_degt_spec = pl.BlockSpec((R, 2), lambda i: (i, 0))
_acc_spec = pl.BlockSpec((2, R, H), lambda i: (0, i, 0))
_lam_spec = pl.BlockSpec((1, 1), lambda i: (0, 0), memory_space=pltpu.SMEM)

_table_out_shapes = [
    jax.ShapeDtypeStruct((N, H), _f32),
    jax.ShapeDtypeStruct((N, H), _f32),
    jax.ShapeDtypeStruct((NP, H), _f32),
    jax.ShapeDtypeStruct((NP, H), _f32),
]
_table_out_specs = [_row_spec(H), _row_spec(H), _row_spec(H), _row_spec(H)]

_pre = pl.pallas_call(
    _pre_body,
    grid=(GRID,),
    in_specs=[_row_spec(H), _degt_spec, _full_spec((H, H)), _full_spec((1, H)),
              _full_spec((H, H)), _full_spec((H, H)), _full_spec((H, H)),
              _full_spec((1, H))],
    out_specs=[_row_spec(H)] + _table_out_specs,
    out_shape=[jax.ShapeDtypeStruct((N, H), _f32)] + _table_out_shapes,
)

_mid = pl.pallas_call(
    _mid_body,
    grid=(GRID,),
    in_specs=[_acc_spec, _acc_spec, _row_spec(H), _row_spec(H), _row_spec(H),
              _degt_spec, _full_spec((H, H)), _full_spec((H, H)),
              _full_spec((H, H)), _full_spec((1, H)), _full_spec((1, H)),
              _lam_spec],
    out_specs=[_row_spec(H)] + _table_out_specs,
    out_shape=[jax.ShapeDtypeStruct((N, H), _f32)] + _table_out_shapes,
)

_post = pl.pallas_call(
    _post_body,
    grid=(GRID,),
    in_specs=[_acc_spec, _acc_spec, _row_spec(H), _row_spec(H), _row_spec(H),
              _degt_spec, _full_spec((NCLASS, H)), _full_spec((1, NCLASS)),
              _full_spec((1, H)), _lam_spec],
    out_specs=[_row_spec(NCLASS)],
    out_shape=[jax.ShapeDtypeStruct((N, NCLASS), _f32)],
)


def kernel(x, edge_index, W_enc, b_enc, W_conv, b_conv, W_res, b_res,
           W_dec, b_dec, weight_mlp, lamda1):
    row = edge_index[0].astype(jnp.int32)
    col = edge_index[1].astype(jnp.int32)
    rowp = jnp.concatenate([row, jnp.arange(PAD, dtype=jnp.int32) % N])
    colp = jnp.concatenate([col, N + (jnp.arange(PAD, dtype=jnp.int32) % 16)])
    zrows = jnp.zeros((NP, H), _f32)

    be = b_enc.reshape(1, H)
    br = b_res.reshape(1, H)
    bc = b_conv.reshape(1, H)
    bd = b_dec.reshape(1, NCLASS)
    lam = lamda1.reshape(1, 1)

    degt = _deg_call(colp).T

    X0, xw1, res1, XWm1, xwn1 = _pre(x, degt, W_enc, be, weight_mlp,
                                     W_conv, W_res, br)
    tp1, sp1 = _edge_call(rowp, colp, XWm1, xwn1, zrows)
    X1, xw2, res2, XWm2, xwn2 = _mid(tp1, sp1, X0, xw1, res1, degt,
                                     weight_mlp, W_conv, W_res, br, bc, lam)
    tp2, sp2 = _edge_call(rowp, colp, XWm2, xwn2, zrows)
    (out,) = _post(tp2, sp2, X1, xw2, res2, degt, W_dec, bd, bc, lam)
    return out
